# baseline (speedup 1.0000x reference)
import jax
import jax.numpy as jnp
from jax.experimental import pallas as pl
from jax.experimental.pallas import tpu as pltpu


_BT = 512
_NBUF = 4


def _gating_manual(x_hbm, mask_ref, w_ref, probs_hbm, logits_hbm,
                   xbuf, pbuf, lbuf, in_sem, pout_sem, lout_sem):
    nsteps = x_hbm.shape[0] // _BT
    w = w_ref[...]

    def in_copy(step, slot):
        return pltpu.make_async_copy(
            x_hbm.at[pl.ds(step * _BT, _BT), :],
            xbuf.at[slot],
            in_sem.at[slot],
        )

    def p_copy(step, slot):
        return pltpu.make_async_copy(
            pbuf.at[slot],
            probs_hbm.at[pl.ds(step * _BT, _BT), :],
            pout_sem.at[slot],
        )

    def l_copy(step, slot):
        return pltpu.make_async_copy(
            lbuf.at[slot],
            logits_hbm.at[pl.ds(step * _BT, _BT), :],
            lout_sem.at[slot],
        )

    for s in range(_NBUF):
        in_copy(s, s).start()

    def body(step, carry):
        slot = jax.lax.rem(step, _NBUF)
        in_copy(step, slot).wait()

        @pl.when(step >= _NBUF)
        def _wait_out():
            p_copy(step - _NBUF, slot).wait()
            l_copy(step - _NBUF, slot).wait()

        logits = jax.lax.dot_general(
            xbuf[slot],
            w,
            dimension_numbers=(((1,), (1,)), ((), ())),
            preferred_element_type=jnp.float32,
        )
        m = jnp.max(logits, axis=-1, keepdims=True)
        e = jnp.exp(logits - m)
        probs = e / jnp.sum(e, axis=-1, keepdims=True)
        pbuf[slot] = probs * mask_ref[pl.ds(step * _BT, _BT), :]
        lbuf[slot] = logits
        p_copy(step, slot).start()
        l_copy(step, slot).start()

        @pl.when(step + _NBUF < nsteps)
        def _next_in():
            in_copy(step + _NBUF, slot).start()

        return carry

    jax.lax.fori_loop(0, nsteps, body, 0)

    for s in range(_NBUF):
        step = nsteps - _NBUF + s
        p_copy(step, step % _NBUF).wait()
        l_copy(step, step % _NBUF).wait()


def kernel(inputs, padding_mask, W):
    T, H = inputs.shape
    E = W.shape[0]
    mask2d = padding_mask.reshape(T, 1)
    probs, logits = pl.pallas_call(
        _gating_manual,
        in_specs=[
            pl.BlockSpec(memory_space=pl.ANY),
            pl.BlockSpec(memory_space=pltpu.VMEM),
            pl.BlockSpec(memory_space=pltpu.VMEM),
        ],
        out_specs=[
            pl.BlockSpec(memory_space=pl.ANY),
            pl.BlockSpec(memory_space=pl.ANY),
        ],
        out_shape=[
            jax.ShapeDtypeStruct((T, E), jnp.float32),
            jax.ShapeDtypeStruct((T, E), jnp.float32),
        ],
        scratch_shapes=[
            pltpu.VMEM((_NBUF, _BT, H), jnp.float32),
            pltpu.VMEM((_NBUF, _BT, E), jnp.float32),
            pltpu.VMEM((_NBUF, _BT, E), jnp.float32),
            pltpu.SemaphoreType.DMA((_NBUF,)),
            pltpu.SemaphoreType.DMA((_NBUF,)),
            pltpu.SemaphoreType.DMA((_NBUF,)),
        ],
    )(inputs, mask2d, W)
    return (probs, logits)


# speedup vs baseline: 1.0371x; 1.0371x over previous
import jax
import jax.numpy as jnp
from jax.experimental import pallas as pl
from jax.experimental.pallas import tpu as pltpu


def _gating_tile(x_ref, mask_ref, w_ref, probs_ref, logits_ref):
    i = pl.program_id(0)
    bt = x_ref.shape[0]
    logits = jax.lax.dot_general(
        x_ref[...],
        w_ref[...],
        dimension_numbers=(((1,), (1,)), ((), ())),
        preferred_element_type=jnp.float32,
    )
    m = jnp.max(logits, axis=-1, keepdims=True)
    e = jnp.exp(logits - m)
    probs = e / jnp.sum(e, axis=-1, keepdims=True)
    probs_ref[...] = probs * mask_ref[pl.ds(i * bt, bt), :]
    logits_ref[...] = logits


def kernel(inputs, padding_mask, W):
    T, H = inputs.shape
    E = W.shape[0]
    BT = 1024
    mask2d = padding_mask.reshape(T, 1)
    probs, logits = pl.pallas_call(
        _gating_tile,
        grid=(T // BT,),
        in_specs=[
            pl.BlockSpec((BT, H), lambda i: (i, 0)),
            pl.BlockSpec((T, 1), lambda i: (0, 0)),
            pl.BlockSpec((E, H), lambda i: (0, 0)),
        ],
        out_specs=[
            pl.BlockSpec((BT, E), lambda i: (i, 0)),
            pl.BlockSpec((BT, E), lambda i: (i, 0)),
        ],
        out_shape=[
            jax.ShapeDtypeStruct((T, E), jnp.float32),
            jax.ShapeDtypeStruct((T, E), jnp.float32),
        ],
        compiler_params=pltpu.CompilerParams(
            dimension_semantics=("parallel",),
        ),
    )(inputs, mask2d, W)
    return (probs, logits)
